# baseline (speedup 1.0000x reference)
import jax
import jax.numpy as jnp
from jax.experimental import pallas as pl
from jax.experimental.pallas import tpu as pltpu

N = 512
B = 32
U = 64
L = 4
NM = 3
IN0 = N + U
INL = 2 * U
GB = 4
GBH = GB // 2
GBN = GB * N
GBNH = GBH * N

_BF16 = jnp.bfloat16


def _dot(a, b):
    return jax.lax.dot_general(a, b, (((1,), (0,)), ((), ())),
                               preferred_element_type=jnp.float32)


def _decoder_kernel(x_ref, h_ref, wx0_ref, whg0_ref, whc0_ref,
                    wx_ref, whg_ref, whc_ref, bg_ref, bc_ref,
                    wp_ref, bp_ref, s_ref, out_ref, hs_ref, t2_ref):
    S = s_ref[...]

    @pl.when(pl.program_id(0) == 0)
    def _():
        ii = jax.lax.broadcasted_iota(jnp.int32, (N, N), 0)
        jj = jax.lax.broadcasted_iota(jnp.int32, (N, N), 1)
        eye = jnp.where(ii == jj, 1.0, 0.0)
        t2_ref[...] = (2.0 * _dot(S, S) - eye).astype(_BF16)

    T2 = t2_ref[...]

    def hops(pb, lo, out):
        p3 = pb.reshape(GBH, N, -1)
        p1c = jnp.concatenate([p3[g][:, lo:lo + out] for g in range(GBH)],
                              axis=1)
        p2c = jnp.concatenate([p3[g][:, lo + out:lo + 2 * out]
                               for g in range(GBH)], axis=1)
        y = _dot(S, p1c) + _dot(T2, p2c)
        return jnp.stack([y[:, g * out:(g + 1) * out] for g in range(GBH)],
                         axis=0).reshape(GBNH, out)

    def cell(xi, h, wx, whg, whc, bgv, bcv):
        xp = _dot(xi, wx)
        pg = xp[:, :384] + _dot(h.astype(_BF16), whg)
        pgb = pg.astype(_BF16)
        val = jax.nn.sigmoid(pg[:, :128] + hops(pgb, 128, 128) + bgv)
        r = val[:, :U]
        u = val[:, U:]
        pc = xp[:, 384:] + _dot((r * h).astype(_BF16), whc)
        pcb = pc.astype(_BF16)
        c = jnp.tanh(pc[:, 128:] + hops(pcb, 0, U) + bcv)
        return u * h + (1.0 - u) * c

    parts = [slice(0, GBH), slice(GBH, GB)]
    hs = [x_ref[p].reshape(GBNH, N).astype(_BF16) for p in parts]
    hs = [cell(hs[i], h_ref[0, parts[i]].reshape(GBNH, U), wx0_ref[...],
               whg0_ref[...], whc0_ref[...], bg_ref[0], bc_ref[0])
          for i in range(2)]
    for i in range(2):
        hs_ref[0, parts[i]] = hs[i].reshape(GBH, N, U)
    for l in range(L - 1):
        hs = [cell(hs[i].astype(_BF16),
                   h_ref[l + 1, parts[i]].reshape(GBNH, U),
                   wx_ref[l], whg_ref[l], whc_ref[l],
                   bg_ref[l + 1], bc_ref[l + 1]) for i in range(2)]
        for i in range(2):
            hs_ref[l + 1, parts[i]] = hs[i].reshape(GBH, N, U)
    for i in range(2):
        proj = _dot(hs[i].astype(_BF16), wp_ref[...]) + bp_ref[...]
        out_ref[parts[i]] = proj.reshape(GBH, N, N)


def _merge_weights(Wgl, Wcl, d):
    wg = Wgl.reshape(d + U, NM, 2 * U).transpose(1, 0, 2)
    wc = Wcl.reshape(d + U, NM, U).transpose(1, 0, 2)
    wx = jnp.concatenate([wg[0, :d], wg[1, :d], wg[2, :d],
                          wc[1, :d], wc[2, :d], wc[0, :d]], axis=1)
    whg = jnp.concatenate([wg[0, d:], wg[1, d:], wg[2, d:]], axis=1)
    whc = jnp.concatenate([wc[1, d:], wc[2, d:], wc[0, d:]], axis=1)
    return wx, whg, whc


def kernel(inputs, hidden_state, Wg0, bg0, Wc0, bc0, Wg, bg, Wc, bc, Wp, bp, support):
    x = inputs.reshape(B, N, N)
    h0 = hidden_state.reshape(L, B, N, U)
    wx0, whg0, whc0 = _merge_weights(Wg0, Wc0, N)
    mw = [_merge_weights(Wg[l], Wc[l], U) for l in range(L - 1)]
    wx = jnp.stack([m[0] for m in mw]).astype(_BF16)
    whg = jnp.stack([m[1] for m in mw]).astype(_BF16)
    whc = jnp.stack([m[2] for m in mw]).astype(_BF16)
    bgall = jnp.concatenate([bg0.reshape(1, 1, 2 * U),
                             bg.reshape(L - 1, 1, 2 * U)], axis=0)
    bcall = jnp.concatenate([bc0.reshape(1, 1, U),
                             bc.reshape(L - 1, 1, U)], axis=0)
    bpr = bp.reshape(1, N)

    out, hs = pl.pallas_call(
        _decoder_kernel,
        grid=(B // GB,),
        in_specs=[
            pl.BlockSpec((GB, N, N), lambda g: (g, 0, 0)),
            pl.BlockSpec((L, GB, N, U), lambda g: (0, g, 0, 0)),
            pl.BlockSpec((N, 9 * U), lambda g: (0, 0)),
            pl.BlockSpec((U, 6 * U), lambda g: (0, 0)),
            pl.BlockSpec((U, 3 * U), lambda g: (0, 0)),
            pl.BlockSpec((L - 1, U, 9 * U), lambda g: (0, 0, 0)),
            pl.BlockSpec((L - 1, U, 6 * U), lambda g: (0, 0, 0)),
            pl.BlockSpec((L - 1, U, 3 * U), lambda g: (0, 0, 0)),
            pl.BlockSpec((L, 1, 2 * U), lambda g: (0, 0, 0)),
            pl.BlockSpec((L, 1, U), lambda g: (0, 0, 0)),
            pl.BlockSpec((U, N), lambda g: (0, 0)),
            pl.BlockSpec((1, N), lambda g: (0, 0)),
            pl.BlockSpec((N, N), lambda g: (0, 0)),
        ],
        out_specs=[
            pl.BlockSpec((GB, N, N), lambda g: (g, 0, 0)),
            pl.BlockSpec((L, GB, N, U), lambda g: (0, g, 0, 0)),
        ],
        out_shape=[
            jax.ShapeDtypeStruct((B, N, N), jnp.float32),
            jax.ShapeDtypeStruct((L, B, N, U), jnp.float32),
        ],
        scratch_shapes=[pltpu.VMEM((N, N), _BF16)],
        compiler_params=pltpu.CompilerParams(
            dimension_semantics=("parallel",),
        ),
    )(x, h0, wx0.astype(_BF16), whg0.astype(_BF16), whc0.astype(_BF16),
      wx, whg, whc, bgall, bcall,
      Wp.astype(_BF16), bpr, support.astype(_BF16))
    return out.reshape(B, N * N), hs.reshape(L, B, N * U)


# speedup vs baseline: 1.1112x; 1.1112x over previous
import jax
import jax.numpy as jnp
from jax.experimental import pallas as pl
from jax.experimental.pallas import tpu as pltpu

N = 512
B = 32
U = 64
L = 4
NM = 3
IN0 = N + U
INL = 2 * U
GB = 4
GBN = GB * N

_BF16 = jnp.bfloat16


def _dot(a, b):
    return jax.lax.dot_general(a, b, (((1,), (0,)), ((), ())),
                               preferred_element_type=jnp.float32)


def _decoder_kernel(x_ref, h_ref, wg0_ref, wc0_ref,
                    wg_ref, wc_ref, bg_ref, bc_ref,
                    wp_ref, bp_ref, s_ref, out_ref, hs_ref, t2_ref):
    S = s_ref[...]

    @pl.when(pl.program_id(0) == 0)
    def _():
        ii = jax.lax.broadcasted_iota(jnp.int32, (N, N), 0)
        jj = jax.lax.broadcasted_iota(jnp.int32, (N, N), 1)
        eye = jnp.where(ii == jj, 1.0, 0.0)
        t2_ref[...] = (2.0 * _dot(S, S) - eye).astype(_BF16)

    T2 = t2_ref[...]

    def hops(pb, lo, out):
        p3 = pb.reshape(GB, N, -1)
        p1c = jnp.concatenate([p3[g][:, lo:lo + out] for g in range(GB)],
                              axis=1)
        p2c = jnp.concatenate([p3[g][:, lo + out:lo + 2 * out]
                               for g in range(GB)], axis=1)
        y = _dot(S, p1c) + _dot(T2, p2c)
        return jnp.stack([y[:, g * out:(g + 1) * out] for g in range(GB)],
                         axis=0).reshape(GBN, out)

    def cell(xi, h, wgf, wcf, bgv, bcv):
        zg = jnp.concatenate([xi, h.astype(_BF16)], axis=1)
        pg = _dot(zg, wgf)
        pgb = pg.astype(_BF16)
        val = jax.nn.sigmoid(pg[:, :128] + hops(pgb, 128, 128) + bgv)
        r = val[:, :U]
        u = val[:, U:]
        zc = jnp.concatenate([xi, (r * h).astype(_BF16)], axis=1)
        pc = _dot(zc, wcf)
        pcb = pc.astype(_BF16)
        c = jnp.tanh(pc[:, :U] + hops(pcb, U, U) + bcv)
        return u * h + (1.0 - u) * c

    xi = x_ref[...].reshape(GBN, N).astype(_BF16)
    h = cell(xi, h_ref[0].reshape(GBN, U), wg0_ref[...], wc0_ref[...],
             bg_ref[0], bc_ref[0])
    hs_ref[0] = h.reshape(GB, N, U)
    for l in range(L - 1):
        h = cell(h.astype(_BF16), h_ref[l + 1].reshape(GBN, U),
                 wg_ref[l], wc_ref[l],
                 bg_ref[l + 1], bc_ref[l + 1])
        hs_ref[l + 1] = h.reshape(GB, N, U)
    proj = _dot(h.astype(_BF16), wp_ref[...]) + bp_ref[...]
    out_ref[...] = proj.reshape(GB, N, N)


def kernel(inputs, hidden_state, Wg0, bg0, Wc0, bc0, Wg, bg, Wc, bc, Wp, bp, support):
    x = inputs.reshape(B, N, N)
    h0 = hidden_state.reshape(L, B, N, U)
    wg0 = Wg0.reshape(IN0, NM * 2 * U)
    wc0 = Wc0.reshape(IN0, NM * U)
    wgl = Wg.reshape(L - 1, INL, NM * 2 * U)
    wcl = Wc.reshape(L - 1, INL, NM * U)
    bgall = jnp.concatenate([bg0.reshape(1, 1, 2 * U),
                             bg.reshape(L - 1, 1, 2 * U)], axis=0)
    bcall = jnp.concatenate([bc0.reshape(1, 1, U),
                             bc.reshape(L - 1, 1, U)], axis=0)
    bpr = bp.reshape(1, N)

    out, hs = pl.pallas_call(
        _decoder_kernel,
        grid=(B // GB,),
        in_specs=[
            pl.BlockSpec((GB, N, N), lambda g: (g, 0, 0)),
            pl.BlockSpec((L, GB, N, U), lambda g: (0, g, 0, 0)),
            pl.BlockSpec((IN0, NM * 2 * U), lambda g: (0, 0)),
            pl.BlockSpec((IN0, NM * U), lambda g: (0, 0)),
            pl.BlockSpec((L - 1, INL, NM * 2 * U), lambda g: (0, 0, 0)),
            pl.BlockSpec((L - 1, INL, NM * U), lambda g: (0, 0, 0)),
            pl.BlockSpec((L, 1, 2 * U), lambda g: (0, 0, 0)),
            pl.BlockSpec((L, 1, U), lambda g: (0, 0, 0)),
            pl.BlockSpec((U, N), lambda g: (0, 0)),
            pl.BlockSpec((1, N), lambda g: (0, 0)),
            pl.BlockSpec((N, N), lambda g: (0, 0)),
        ],
        out_specs=[
            pl.BlockSpec((GB, N, N), lambda g: (g, 0, 0)),
            pl.BlockSpec((L, GB, N, U), lambda g: (0, g, 0, 0)),
        ],
        out_shape=[
            jax.ShapeDtypeStruct((B, N, N), jnp.float32),
            jax.ShapeDtypeStruct((L, B, N, U), jnp.float32),
        ],
        scratch_shapes=[pltpu.VMEM((N, N), _BF16)],
        compiler_params=pltpu.CompilerParams(
            dimension_semantics=("parallel",),
        ),
    )(x, h0, wg0.astype(_BF16), wc0.astype(_BF16),
      wgl.astype(_BF16), wcl.astype(_BF16), bgall, bcall,
      Wp.astype(_BF16), bpr, support.astype(_BF16))
    return out.reshape(B, N * N), hs.reshape(L, B, N * U)
